# baseline (speedup 1.0000x reference)
import functools

import jax
import jax.numpy as jnp
from jax import lax
from jax.experimental import pallas as pl
from jax.experimental.pallas import tpu as pltpu
from jax.experimental.pallas import tpu_sc as plsc

_INFO = plsc.get_sparse_core_info()
_NC = _INFO.num_cores
_NS = _INFO.num_subcores
_NW = _NC * _NS


@functools.lru_cache(maxsize=None)
def _make_gather(batch, vocab, dim):
    assert batch % (8 * _NW) == 0
    b_per_w = batch // _NW
    mesh = plsc.VectorSubcoreMesh(core_axis_name="c", subcore_axis_name="s")

    @functools.partial(
        pl.kernel,
        mesh=mesh,
        out_type=jax.ShapeDtypeStruct((batch, dim), jnp.float32),
        compiler_params=pltpu.CompilerParams(use_tc_tiling_on_sc=False),
        scratch_types=[
            pltpu.VMEM((b_per_w,), jnp.int32),
            pltpu.VMEM((b_per_w, dim), jnp.float32),
            pltpu.SemaphoreType.DMA,
        ],
    )
    def gather(table_hbm, idx_hbm, out_hbm, idx_v, rows_v, sem):
        wid = lax.axis_index("s") * _NC + lax.axis_index("c")
        base = wid * b_per_w
        pltpu.sync_copy(idx_hbm.at[pl.ds(base, b_per_w)], idx_v)
        pltpu.async_copy(table_hbm.at[idx_v], rows_v, sem).wait()
        pltpu.sync_copy(rows_v, out_hbm.at[pl.ds(base, b_per_w)])

    return gather


@jax.jit
def kernel(codes, emb_weight):
    batch, = codes.shape
    vocab, dim = emb_weight.shape
    gather = _make_gather(batch, vocab, dim)
    return gather(emb_weight, codes.astype(jnp.int32))


# speedup vs baseline: 1.6324x; 1.6324x over previous
import functools

import jax
import jax.numpy as jnp
from jax import lax
from jax.experimental import pallas as pl
from jax.experimental.pallas import tpu as pltpu
from jax.experimental.pallas import tpu_sc as plsc

_INFO = plsc.get_sparse_core_info()
_NC = _INFO.num_cores
_NS = _INFO.num_subcores
_NW = _NC * _NS

_GRP = 16


@functools.lru_cache(maxsize=None)
def _make_gather(batch, vocab, dim):
    assert batch % (8 * _NW) == 0 and dim == 64
    b_per_w = batch // _NW
    mesh = plsc.VectorSubcoreMesh(core_axis_name="c", subcore_axis_name="s")

    @functools.partial(
        pl.kernel,
        mesh=mesh,
        out_type=jax.ShapeDtypeStruct((batch, dim), jnp.float32),
        scratch_types=[
            pltpu.VMEM((b_per_w,), jnp.int32),
            pltpu.VMEM((b_per_w, dim), jnp.float32),
            pltpu.SemaphoreType.DMA,
        ],
    )
    def gather(table_hbm, idx_hbm, out_hbm, idx_v, rows_v, sem):
        wid = lax.axis_index("s") * _NC + lax.axis_index("c")
        base = wid * b_per_w
        pltpu.sync_copy(idx_hbm.at[pl.ds(base, b_per_w)], idx_v)

        def group(g, _):
            i0 = g * _GRP
            idx_vec = idx_v[pl.ds(i0, _GRP)]
            copies = []
            for j in range(_GRP):
                r = idx_vec[j]
                copies.append(
                    pltpu.async_copy(
                        table_hbm.at[pl.ds(r, 1), :],
                        rows_v.at[pl.ds(i0 + j, 1), :],
                        sem,
                    )
                )
            for c in copies:
                c.wait()
            return ()

        lax.fori_loop(0, b_per_w // _GRP, group, (), unroll=False)
        pltpu.sync_copy(rows_v, out_hbm.at[pl.ds(base, b_per_w)])

    return gather


@jax.jit
def kernel(codes, emb_weight):
    batch, = codes.shape
    vocab, dim = emb_weight.shape
    gather = _make_gather(batch, vocab, dim)
    return gather(emb_weight, codes.astype(jnp.int32))


# speedup vs baseline: 2.1264x; 1.3026x over previous
import functools

import jax
import jax.numpy as jnp
from jax import lax
from jax.experimental import pallas as pl
from jax.experimental.pallas import tpu as pltpu
from jax.experimental.pallas import tpu_sc as plsc

_INFO = plsc.get_sparse_core_info()
_NC = _INFO.num_cores
_NS = _INFO.num_subcores
_NW = _NC * _NS

_GRP = 16
_TCHUNK = 15872


@functools.lru_cache(maxsize=None)
def _make_transpose(vocab, dim):
    main = (vocab // _TCHUNK) * _TCHUNK

    def body(x_ref, o_ref):
        o_ref[...] = x_ref[...].T

    return pl.pallas_call(
        body,
        grid=(main // _TCHUNK,),
        in_specs=[
            pl.BlockSpec((dim, _TCHUNK), lambda g: (0, g)),
        ],
        out_specs=pl.BlockSpec((_TCHUNK, dim), lambda g: (g, 0)),
        out_shape=jax.ShapeDtypeStruct((vocab, dim), jnp.float32),
    )


@functools.lru_cache(maxsize=None)
def _make_gather(batch, vocab, dim):
    assert batch % (8 * _NW) == 0 and dim == 64
    b_per_w = batch // _NW
    mesh = plsc.VectorSubcoreMesh(core_axis_name="c", subcore_axis_name="s")

    @functools.partial(
        pl.kernel,
        mesh=mesh,
        out_type=jax.ShapeDtypeStruct((batch, dim), jnp.float32),
        scratch_types=[
            pltpu.VMEM((b_per_w,), jnp.int32),
            pltpu.VMEM((b_per_w, dim), jnp.float32),
            pltpu.SemaphoreType.DMA,
        ],
    )
    def gather(table_hbm, idx_hbm, out_hbm, idx_v, rows_v, sem):
        wid = lax.axis_index("s") * _NC + lax.axis_index("c")
        base = wid * b_per_w
        pltpu.sync_copy(idx_hbm.at[pl.ds(base, b_per_w)], idx_v)

        def group(g, _):
            i0 = g * _GRP
            idx_vec = idx_v[pl.ds(i0, _GRP)]
            copies = []
            for j in range(_GRP):
                r = idx_vec[j]
                copies.append(
                    pltpu.async_copy(
                        table_hbm.at[pl.ds(r, 1), :],
                        rows_v.at[pl.ds(i0 + j, 1), :],
                        sem,
                    )
                )
            for c in copies:
                c.wait()
            return ()

        lax.fori_loop(0, b_per_w // _GRP, group, (), unroll=False)
        pltpu.sync_copy(rows_v, out_hbm.at[pl.ds(base, b_per_w)])

    return gather


@jax.jit
def kernel(codes, emb_weight):
    batch, = codes.shape
    vocab, dim = emb_weight.shape
    main = (vocab // _TCHUNK) * _TCHUNK
    table_rm = _make_transpose(vocab, dim)(emb_weight.T)
    if main != vocab:
        table_rm = lax.dynamic_update_slice(
            table_rm, emb_weight[main:, :], (main, 0)
        )
    gather = _make_gather(batch, vocab, dim)
    return gather(table_rm, codes.astype(jnp.int32))
